# baseline (speedup 1.0000x reference)
import functools

import numpy as np

import jax
import jax.numpy as jnp
from jax.experimental import pallas as pl
from jax.experimental.pallas import tpu as pltpu

_CB = 8192
_D = 32
_B, _T, _H, _W = 4, 4, 32, 32
_HW = _H * _W
_NROWS = _T * _B * _HW

_HW_SCALES = [(1, 1), (2, 2), (3, 3), (4, 4), (6, 6), (9, 9), (13, 13),
              (18, 18), (24, 24), (32, 32)]
_T_SCALES = [1, 2, 3, 4, 5, 6, 7, 9, 11, 13]
_SCHEDULE = [(min(_T, ts), hh, ww) for (hh, ww), ts in zip(_HW_SCALES, _T_SCALES)]
_NSCALES = len(_SCHEDULE)
_PT = [pt for (pt, ph, pw) in _SCHEDULE]
_PHW = [ph * pw for (pt, ph, pw) in _SCHEDULE]
_PHW8 = [-(-p // 8) * 8 for p in _PHW]
_NBLK = [_B * pt for pt in _PT]
_NPAD = [nb * p8 for nb, p8 in zip(_NBLK, _PHW8)]
_TILE = 1024
_CH = 1024
_NT = [-(-n // _TILE) for n in _NPAD]


def _weight_mat(n_in, n_out):
    scale = n_out / n_in
    inv_scale = 1.0 / scale
    kernel_scale = max(inv_scale, 1.0)
    sample_f = (np.arange(n_out, dtype=np.float64) + 0.5) * inv_scale - 0.5
    x = np.abs(sample_f[None, :] - np.arange(n_in, dtype=np.float64)[:, None]) / kernel_scale
    w = np.maximum(0.0, 1.0 - x)
    total = w.sum(axis=0, keepdims=True)
    w = np.where(np.abs(total) > 1000 * np.finfo(np.float32).eps,
                 w / np.where(total != 0, total, 1), 0.0)
    w = w * ((sample_f >= -0.5) & (sample_f <= n_in - 0.5))[None, :]
    return w.astype(np.float32)


_MD = []
_MUT = []
_DT = []
_UT = []
for (_pt, _ph, _pw) in _SCHEDULE[:-1]:
    _wdh = _weight_mat(_H, _ph)
    _wdw = _weight_mat(_W, _pw)
    _MD.append(np.ascontiguousarray(np.kron(_wdh, _wdw).T))
    _wuh = _weight_mat(_ph, _H)
    _wuw = _weight_mat(_pw, _W)
    _MUT.append(np.ascontiguousarray(np.kron(_wuh, _wuw)))
    _DT.append([[float(v) for v in row] for row in _weight_mat(_T, _pt).T])
    _UT.append([[float(v) for v in row] for row in _weight_mat(_pt, _T).T])

_F32 = jnp.float32
_HI = jax.lax.Precision.HIGHEST


def _mm(a, b, contract, precision=None):
    return jax.lax.dot_general(a, b, (contract, ((), ())),
                               preferred_element_type=_F32,
                               precision=precision)


def _prep_kernel(cbt_ref, c1_ref, c2_ref, c3_ref):
    cb = cbt_ref[...]
    norm = jnp.sqrt(jnp.sum(cb * cb, axis=0, keepdims=True))
    cbn = cb / jnp.maximum(norm, 1e-12)
    c1 = cbn.astype(jnp.bfloat16).astype(_F32)
    rem = cbn - c1
    c2 = rem.astype(jnp.bfloat16).astype(_F32)
    c1_ref[...] = c1
    c2_ref[...] = c2
    c3_ref[...] = rem - c2


def _prep(cbt):
    shp = jax.ShapeDtypeStruct((_D, _CB), _F32)
    return pl.pallas_call(_prep_kernel, out_shape=[shp, shp, shp])(cbt)


def _step_kernel(s, *refs):
    i = 0
    if s > 0:
        qprev_ref = refs[i]
        mut_ref = refs[i + 1]
        i += 2
    r_ref = refs[i]
    i += 1
    if s < _NSCALES - 1:
        md_ref = refs[i]
        i += 1
    rout_ref, tok_ref = refs[i], refs[i + 1]

    rout_ref[...] = r_ref[...]

    if s > 0:
        sp = s - 1
        ptp, phwp, phw8p = _PT[sp], _PHW[sp], _PHW8[sp]

        def up_body(bi, carry):
            mut = mut_ref[...]
            qh = [_mm(mut, qprev_ref[pl.ds((bi * ptp + k) * phw8p, phwp), :],
                      ((0,), (0,)), _HI)
                  for k in range(ptp)]
            for t in range(_T):
                if ptp == _T:
                    full = qh[t]
                else:
                    full = sum(_UT[sp][t][k] * qh[k] for k in range(ptp)
                               if _UT[sp][t][k] != 0.0)
                off = (t * _B + bi) * _HW
                rout_ref[pl.ds(off, _HW), :] -= full
            return carry
        jax.lax.fori_loop(0, _B, up_body, 0)

    if s == _NSCALES - 1:
        def tok_last(kb, carry):
            bi, t = kb // _T, kb % _T
            tok_ref[pl.ds(kb * _HW, _HW), :] = (
                rout_ref[pl.ds((t * _B + bi) * _HW, _HW), :])
            return carry
        jax.lax.fori_loop(0, _B * _T, tok_last, 0)
    else:
        pt, phw, phw8 = _PT[s], _PHW[s], _PHW8[s]

        def down_body(bi, carry):
            md = md_ref[...]
            rblk = [rout_ref[pl.ds((t * _B + bi) * _HW, _HW), :]
                    for t in range(_T)]
            for k in range(pt):
                if pt == _T:
                    yt = rblk[k]
                else:
                    yt = sum(_DT[s][k][t] * rblk[t] for t in range(_T)
                             if _DT[s][k][t] != 0.0)
                tok = _mm(md, yt, ((1,), (0,)), _HI)
                tok_ref[pl.ds((bi * pt + k) * phw8, phw), :] = tok
            return carry
        jax.lax.fori_loop(0, _B, down_body, 0)


def _step(s, r, q_prev, mats):
    args = []
    if s > 0:
        args += [q_prev, mats[1][s - 1]]
        in_idx = 2
    else:
        in_idx = 0
    args.append(r)
    if s < _NSCALES - 1:
        args.append(mats[0][s])
    out_shape = [jax.ShapeDtypeStruct((_NROWS, _D), _F32),
                 jax.ShapeDtypeStruct((_NT[s] * _TILE, _D), _F32)]
    return pl.pallas_call(
        functools.partial(_step_kernel, s),
        out_shape=out_shape,
        input_output_aliases={in_idx: 0},
    )(*args)


def _vq_kernel(tok_ref, c1_ref, c2_ref, c3_ref, idx_ref, q_ref):
    toks = tok_ref[...]
    iota = jax.lax.broadcasted_iota(jnp.int32, (_TILE, _CH), 1)

    best = jnp.full((_TILE, 1), -jnp.inf, _F32)
    bidx = jnp.full((_TILE, 1), _CB, jnp.int32)
    for c in range(_CB // _CH):
        cb_c = c1_ref[:, pl.ds(c * _CH, _CH)]
        d = _mm(toks, cb_c, ((1,), (0,)))
        m = jnp.max(d, axis=1, keepdims=True)
        cidx = jnp.min(jnp.where(d == m, iota, _CB), axis=1,
                       keepdims=True) + c * _CH
        upd = m > best
        best = jnp.where(upd, m, best)
        bidx = jnp.where(upd, cidx, bidx)

    q = jnp.zeros((_TILE, _D), _F32)
    for c in range(_CB // _CH):
        oh = (iota == bidx - c * _CH).astype(_F32)
        for cref in (c1_ref, c2_ref, c3_ref):
            cb_c = cref[:, pl.ds(c * _CH, _CH)]
            q = q + _mm(oh, cb_c, ((1,), (1,)))
    q_ref[...] = q
    idx_ref[...] = bidx.reshape(1, 1, _TILE)


def _vq(s, tok, comps):
    nt = _NT[s]
    full = pl.BlockSpec((_D, _CB), lambda i: (0, 0))
    return pl.pallas_call(
        _vq_kernel,
        grid=(nt,),
        in_specs=[pl.BlockSpec((_TILE, _D), lambda i: (i, 0)),
                  full, full, full],
        out_specs=[pl.BlockSpec((1, 1, _TILE), lambda i: (i, 0, 0)),
                   pl.BlockSpec((_TILE, _D), lambda i: (i, 0))],
        out_shape=[jax.ShapeDtypeStruct((nt, 1, _TILE), jnp.int32),
                   jax.ShapeDtypeStruct((nt * _TILE, _D), _F32)],
        compiler_params=pltpu.CompilerParams(
            dimension_semantics=("parallel",)),
    )(tok, *comps)


def _final_kernel(z_ref, r_ref, qlast_ref, out_ref):
    def body(kb, carry):
        bi, t = kb // _T, kb % _T
        off = (t * _B + bi) * _HW
        out_ref[pl.ds(off, _HW), :] = (
            z_ref[pl.ds(off, _HW), :] - r_ref[pl.ds(off, _HW), :]
            + qlast_ref[pl.ds(kb * _HW, _HW), :])
        return carry
    jax.lax.fori_loop(0, _B * _T, body, 0)


def _final(z2, r, q_last):
    return pl.pallas_call(
        _final_kernel,
        out_shape=jax.ShapeDtypeStruct((_NROWS, _D), _F32),
    )(z2, r, q_last)


def kernel(z, codebook):
    z2 = z.transpose(2, 0, 3, 4, 1).reshape(_NROWS, _D)
    mats = ([jnp.asarray(m) for m in _MD], [jnp.asarray(m) for m in _MUT])
    comps = _prep(codebook.T)

    r = z2
    q_prev = None
    idxs = []
    for s in range(_NSCALES):
        r, tok = _step(s, r, q_prev, mats)
        idx_s, q_prev = _vq(s, tok, comps)
        idxs.append(idx_s)
    qout2 = _final(z2, r, q_prev)

    qout = qout2.reshape(_T, _B, _H, _W, _D).transpose(1, 4, 0, 2, 3)
    idx = jnp.concatenate(
        [o.reshape(-1)[:npad].reshape(nb, p8)[:, :p].reshape(-1)
         for o, npad, nb, p8, p in zip(idxs, _NPAD, _NBLK, _PHW8, _PHW)])
    return qout, idx


# speedup vs baseline: 1.0256x; 1.0256x over previous
import functools

import numpy as np

import jax
import jax.numpy as jnp
from jax.experimental import pallas as pl
from jax.experimental.pallas import tpu as pltpu

_CB = 8192
_D = 32
_B, _T, _H, _W = 4, 4, 32, 32
_HW = _H * _W
_NROWS = _T * _B * _HW

_HW_SCALES = [(1, 1), (2, 2), (3, 3), (4, 4), (6, 6), (9, 9), (13, 13),
              (18, 18), (24, 24), (32, 32)]
_T_SCALES = [1, 2, 3, 4, 5, 6, 7, 9, 11, 13]
_SCHEDULE = [(min(_T, ts), hh, ww) for (hh, ww), ts in zip(_HW_SCALES, _T_SCALES)]
_NSCALES = len(_SCHEDULE)
_PT = [pt for (pt, ph, pw) in _SCHEDULE]
_PHW = [ph * pw for (pt, ph, pw) in _SCHEDULE]
_PHW8 = [-(-p // 8) * 8 for p in _PHW]
_NBLK = [_B * pt for pt in _PT]
_NPAD = [nb * p8 for nb, p8 in zip(_NBLK, _PHW8)]
_TILE = 1024
_CH = 1024
_NT = [-(-n // _TILE) for n in _NPAD]


def _weight_mat(n_in, n_out):
    scale = n_out / n_in
    inv_scale = 1.0 / scale
    kernel_scale = max(inv_scale, 1.0)
    sample_f = (np.arange(n_out, dtype=np.float64) + 0.5) * inv_scale - 0.5
    x = np.abs(sample_f[None, :] - np.arange(n_in, dtype=np.float64)[:, None]) / kernel_scale
    w = np.maximum(0.0, 1.0 - x)
    total = w.sum(axis=0, keepdims=True)
    w = np.where(np.abs(total) > 1000 * np.finfo(np.float32).eps,
                 w / np.where(total != 0, total, 1), 0.0)
    w = w * ((sample_f >= -0.5) & (sample_f <= n_in - 0.5))[None, :]
    return w.astype(np.float32)


_MD = []
_MUT = []
_DT = []
_UT = []
for (_pt, _ph, _pw) in _SCHEDULE[:-1]:
    _wdh = _weight_mat(_H, _ph)
    _wdw = _weight_mat(_W, _pw)
    _MD.append(np.ascontiguousarray(np.kron(_wdh, _wdw).T))
    _wuh = _weight_mat(_ph, _H)
    _wuw = _weight_mat(_pw, _W)
    _MUT.append(np.ascontiguousarray(np.kron(_wuh, _wuw)))
    _DT.append([[float(v) for v in row] for row in _weight_mat(_T, _pt).T])
    _UT.append([[float(v) for v in row] for row in _weight_mat(_pt, _T).T])

_F32 = jnp.float32
_HI = jax.lax.Precision.HIGHEST


def _mm(a, b, contract, precision=None):
    return jax.lax.dot_general(a, b, (contract, ((), ())),
                               preferred_element_type=_F32,
                               precision=precision)


def _prep_kernel(cbt_ref, c1_ref, c2_ref, c3_ref):
    cb = cbt_ref[...]
    norm = jnp.sqrt(jnp.sum(cb * cb, axis=0, keepdims=True))
    cbn = cb / jnp.maximum(norm, 1e-12)
    c1 = cbn.astype(jnp.bfloat16).astype(_F32)
    rem = cbn - c1
    c2 = rem.astype(jnp.bfloat16).astype(_F32)
    c1_ref[...] = c1
    c2_ref[...] = c2
    c3_ref[...] = rem - c2


def _prep(cbt):
    shp = jax.ShapeDtypeStruct((_D, _CB), _F32)
    return pl.pallas_call(_prep_kernel, out_shape=[shp, shp, shp])(cbt)


def _step_kernel(s, *refs):
    i = 0
    if s > 0:
        qprev_ref = refs[i]
        mut_ref = refs[i + 1]
        i += 2
    r_ref = refs[i]
    i += 1
    if s < _NSCALES - 1:
        md_ref = refs[i]
        i += 1
    rout_ref, tok_ref = refs[i], refs[i + 1]

    rout_ref[...] = r_ref[...]

    if s > 0:
        sp = s - 1
        ptp, phwp, phw8p = _PT[sp], _PHW[sp], _PHW8[sp]

        def up_body(bi, carry):
            mut = mut_ref[...]
            qh = [_mm(mut, qprev_ref[pl.ds((bi * ptp + k) * phw8p, phwp), :],
                      ((0,), (0,)), _HI)
                  for k in range(ptp)]
            for t in range(_T):
                if ptp == _T:
                    full = qh[t]
                else:
                    full = sum(_UT[sp][t][k] * qh[k] for k in range(ptp)
                               if _UT[sp][t][k] != 0.0)
                off = (t * _B + bi) * _HW
                rout_ref[pl.ds(off, _HW), :] -= full
            return carry
        jax.lax.fori_loop(0, _B, up_body, 0)

    if s == _NSCALES - 1:
        def tok_last(kb, carry):
            bi, t = kb // _T, kb % _T
            tok_ref[pl.ds(kb * _HW, _HW), :] = (
                rout_ref[pl.ds((t * _B + bi) * _HW, _HW), :])
            return carry
        jax.lax.fori_loop(0, _B * _T, tok_last, 0)
    else:
        pt, phw, phw8 = _PT[s], _PHW[s], _PHW8[s]

        def down_body(bi, carry):
            md = md_ref[...]
            rblk = [rout_ref[pl.ds((t * _B + bi) * _HW, _HW), :]
                    for t in range(_T)]
            for k in range(pt):
                if pt == _T:
                    yt = rblk[k]
                else:
                    yt = sum(_DT[s][k][t] * rblk[t] for t in range(_T)
                             if _DT[s][k][t] != 0.0)
                tok = _mm(md, yt, ((1,), (0,)), _HI)
                tok_ref[pl.ds((bi * pt + k) * phw8, phw), :] = tok
            return carry
        jax.lax.fori_loop(0, _B, down_body, 0)


def _step(s, r, q_prev, mats):
    args = []
    if s > 0:
        args += [q_prev, mats[1][s - 1]]
        in_idx = 2
    else:
        in_idx = 0
    args.append(r)
    if s < _NSCALES - 1:
        args.append(mats[0][s])
    out_shape = [jax.ShapeDtypeStruct((_NROWS, _D), _F32),
                 jax.ShapeDtypeStruct((_NT[s] * _TILE, _D), _F32)]
    return pl.pallas_call(
        functools.partial(_step_kernel, s),
        out_shape=out_shape,
        input_output_aliases={in_idx: 0},
    )(*args)


def _vq_kernel(tok_ref, c1_ref, c2_ref, c3_ref, idx_ref, q_ref):
    toks = tok_ref[...]
    iota = jax.lax.broadcasted_iota(jnp.int32, (_TILE, _CH), 1)

    def scan_chunk(c, carry):
        best, bidx = carry
        cb_c = c1_ref[:, pl.ds(c * _CH, _CH)]
        d = _mm(toks, cb_c, ((1,), (0,)))
        m = jnp.max(d, axis=1, keepdims=True)
        cidx = jnp.min(jnp.where(d == m, iota, _CB), axis=1,
                       keepdims=True) + c * _CH
        upd = m > best
        return jnp.where(upd, m, best), jnp.where(upd, cidx, bidx)

    best0 = jnp.full((_TILE, 1), -jnp.inf, _F32)
    bidx0 = jnp.full((_TILE, 1), _CB, jnp.int32)
    _, bidx = jax.lax.fori_loop(0, _CB // _CH, scan_chunk, (best0, bidx0))

    def gather_chunk(c, q):
        oh = (iota == bidx - c * _CH).astype(_F32)
        for cref in (c1_ref, c2_ref, c3_ref):
            cb_c = cref[:, pl.ds(c * _CH, _CH)]
            q = q + _mm(oh, cb_c, ((1,), (1,)))
        return q

    q = jax.lax.fori_loop(0, _CB // _CH, gather_chunk,
                          jnp.zeros((_TILE, _D), _F32))
    q_ref[...] = q
    idx_ref[...] = bidx.reshape(1, 1, _TILE)


def _vq(s, tok, comps):
    nt = _NT[s]
    full = pl.BlockSpec((_D, _CB), lambda i: (0, 0))
    return pl.pallas_call(
        _vq_kernel,
        grid=(nt,),
        in_specs=[pl.BlockSpec((_TILE, _D), lambda i: (i, 0)),
                  full, full, full],
        out_specs=[pl.BlockSpec((1, 1, _TILE), lambda i: (i, 0, 0)),
                   pl.BlockSpec((_TILE, _D), lambda i: (i, 0))],
        out_shape=[jax.ShapeDtypeStruct((nt, 1, _TILE), jnp.int32),
                   jax.ShapeDtypeStruct((nt * _TILE, _D), _F32)],
        compiler_params=pltpu.CompilerParams(
            dimension_semantics=("parallel",)),
    )(tok, *comps)


def _final_kernel(z_ref, r_ref, qlast_ref, out_ref):
    def body(kb, carry):
        bi, t = kb // _T, kb % _T
        off = (t * _B + bi) * _HW
        out_ref[pl.ds(off, _HW), :] = (
            z_ref[pl.ds(off, _HW), :] - r_ref[pl.ds(off, _HW), :]
            + qlast_ref[pl.ds(kb * _HW, _HW), :])
        return carry
    jax.lax.fori_loop(0, _B * _T, body, 0)


def _final(z2, r, q_last):
    return pl.pallas_call(
        _final_kernel,
        out_shape=jax.ShapeDtypeStruct((_NROWS, _D), _F32),
    )(z2, r, q_last)


def kernel(z, codebook):
    z2 = z.transpose(2, 0, 3, 4, 1).reshape(_NROWS, _D)
    mats = ([jnp.asarray(m) for m in _MD], [jnp.asarray(m) for m in _MUT])
    comps = _prep(codebook.T)

    r = z2
    q_prev = None
    idxs = []
    for s in range(_NSCALES):
        r, tok = _step(s, r, q_prev, mats)
        idx_s, q_prev = _vq(s, tok, comps)
        idxs.append(idx_s)
    qout2 = _final(z2, r, q_prev)

    qout = qout2.reshape(_T, _B, _H, _W, _D).transpose(1, 4, 0, 2, 3)
    idx = jnp.concatenate(
        [o.reshape(-1)[:npad].reshape(nb, p8)[:, :p].reshape(-1)
         for o, npad, nb, p8, p in zip(idxs, _NPAD, _NBLK, _PHW8, _PHW)])
    return qout, idx
